# Initial kernel scaffold; baseline (speedup 1.0000x reference)
import jax
import jax.numpy as jnp
from jax.experimental import pallas as pl


def kernel(y, j, sub, A, U, Lambda):
    raise NotImplementedError("write your pallas kernel here")


# speedup vs baseline: 145.8453x; 145.8453x over previous
import functools

import jax
import jax.numpy as jnp
from jax import lax
from jax.experimental import pallas as pl
from jax.experimental.pallas import tpu as pltpu
from jax.experimental.pallas import tpu_sc as plsc

N_OBS = 16384 * 200
NUM_CORES = 2
NUM_SUBCORES = 16
NW = NUM_CORES * NUM_SUBCORES
PER_W = N_OBS // NW
CHUNK = 2048
LANES = 16


def _loss_partials(y, j, sub, A, U, Lambda):
    mesh = plsc.VectorSubcoreMesh(core_axis_name="c", subcore_axis_name="s")

    @functools.partial(
        pl.kernel,
        out_type=jax.ShapeDtypeStruct((NW, LANES), jnp.float32),
        mesh=mesh,
        scratch_types=[
            pltpu.VMEM((CHUNK,), jnp.float32),
            pltpu.VMEM((CHUNK,), jnp.float32),
            pltpu.VMEM((CHUNK,), jnp.int32),
            pltpu.VMEM((CHUNK,), jnp.float32),
            pltpu.VMEM((CHUNK,), jnp.float32),
            pltpu.VMEM((CHUNK,), jnp.float32),
            pltpu.VMEM((LANES,), jnp.float32),
            pltpu.SemaphoreType.DMA,
        ],
    )
    def k(y_hbm, j_hbm, sub_hbm, a_hbm, u_hbm, l_hbm, out_hbm,
          y_v, j_v, idx_v, a_v, u_v, l_v, acc_v, sem):
        wid = lax.axis_index("s") * NUM_CORES + lax.axis_index("c")
        base = wid * PER_W

        def chunk_body(i, acc):
            off = base + i * CHUNK
            pltpu.sync_copy(y_hbm.at[pl.ds(off, CHUNK)], y_v)
            pltpu.sync_copy(j_hbm.at[pl.ds(off, CHUNK)], j_v)
            pltpu.sync_copy(sub_hbm.at[pl.ds(off, CHUNK)], idx_v)
            pltpu.async_copy(a_hbm.at[idx_v], a_v, sem).wait()
            pltpu.async_copy(u_hbm.at[idx_v], u_v, sem).wait()
            pltpu.async_copy(l_hbm.at[idx_v], l_v, sem).wait()

            def vec_body(v, acc):
                sl = pl.ds(v * LANES, LANES)
                a = jnp.maximum(a_v[sl], 0.0)
                u = jnp.maximum(u_v[sl], 0.0)
                lam = l_v[sl]
                rate = 0.2 / (1.0 + jnp.exp(-lam))
                mu = a - u * jnp.exp(-rate * j_v[sl])
                r = y_v[sl] - mu
                return acc + r * r

            return lax.fori_loop(0, CHUNK // LANES, vec_body, acc)

        acc = lax.fori_loop(0, PER_W // CHUNK, chunk_body,
                            jnp.zeros((LANES,), jnp.float32))
        acc_v[...] = acc
        pltpu.sync_copy(acc_v, out_hbm.at[wid])

    return k(y, j, sub, A, U, Lambda)


def kernel(y, j, sub, A, U, Lambda):
    partials = _loss_partials(y, j, sub, A, U, Lambda)
    return jnp.sqrt(jnp.sum(partials) / N_OBS)


# speedup vs baseline: 203.5670x; 1.3958x over previous
import functools

import jax
import jax.numpy as jnp
from jax import lax
from jax.experimental import pallas as pl
from jax.experimental.pallas import tpu as pltpu
from jax.experimental.pallas import tpu_sc as plsc

N_OBS = 16384 * 200
NUM_CORES = 2
NUM_SUBCORES = 16
NW = NUM_CORES * NUM_SUBCORES
PER_W = N_OBS // NW
CHUNK = 2048
NCHUNK = PER_W // CHUNK
LANES = 16

SUBJECTS_PAD = 100352
ROWS_W = SUBJECTS_PAD // NW

_PARAMS = pltpu.CompilerParams(needs_layout_passes=False)


def _transform_tables(A, U, Lambda):
    mesh = plsc.VectorSubcoreMesh(core_axis_name="c", subcore_axis_name="s")

    @functools.partial(
        pl.kernel,
        out_type=(pltpu.HBM((SUBJECTS_PAD,), jnp.float32),
                  pltpu.HBM((SUBJECTS_PAD,), jnp.float32),
                  pltpu.HBM((SUBJECTS_PAD,), jnp.float32)),
        mesh=mesh,
        compiler_params=_PARAMS,
        scratch_types=[
            pltpu.VMEM((ROWS_W,), jnp.float32),
            pltpu.VMEM((ROWS_W,), jnp.float32),
            pltpu.VMEM((ROWS_W,), jnp.float32),
        ],
    )
    def k(a_hbm, u_hbm, l_hbm, ao_hbm, uo_hbm, ro_hbm, a_v, u_v, l_v):
        wid = lax.axis_index("s") * NUM_CORES + lax.axis_index("c")
        base = wid * ROWS_W
        pltpu.sync_copy(a_hbm.at[pl.ds(base, ROWS_W)], a_v)
        pltpu.sync_copy(u_hbm.at[pl.ds(base, ROWS_W)], u_v)
        pltpu.sync_copy(l_hbm.at[pl.ds(base, ROWS_W)], l_v)

        def body(v, carry):
            sl = pl.ds(v * LANES, LANES)
            a_v[sl] = jnp.maximum(a_v[sl], 0.0)
            u_v[sl] = jnp.maximum(u_v[sl], 0.0)
            l_v[sl] = 0.2 / (1.0 + jnp.exp(-l_v[sl]))
            return carry

        lax.fori_loop(0, ROWS_W // LANES, body, 0)
        pltpu.sync_copy(a_v, ao_hbm.at[pl.ds(base, ROWS_W)])
        pltpu.sync_copy(u_v, uo_hbm.at[pl.ds(base, ROWS_W)])
        pltpu.sync_copy(l_v, ro_hbm.at[pl.ds(base, ROWS_W)])

    return k(A, U, Lambda)


def _loss_partials(y, j, sub, a_tab, u_tab, r_tab):
    mesh = plsc.VectorSubcoreMesh(core_axis_name="c", subcore_axis_name="s")

    @functools.partial(
        pl.kernel,
        out_type=jax.ShapeDtypeStruct((NW, LANES), jnp.float32),
        mesh=mesh,
        compiler_params=_PARAMS,
        scratch_types=[
            [pltpu.VMEM((CHUNK,), jnp.float32) for _ in range(2)],
            [pltpu.VMEM((CHUNK,), jnp.float32) for _ in range(2)],
            [pltpu.VMEM((CHUNK,), jnp.int32) for _ in range(2)],
            [pltpu.VMEM((CHUNK,), jnp.float32) for _ in range(2)],
            [pltpu.VMEM((CHUNK,), jnp.float32) for _ in range(2)],
            [pltpu.VMEM((CHUNK,), jnp.float32) for _ in range(2)],
            [pltpu.SemaphoreType.DMA for _ in range(2)],
            [pltpu.SemaphoreType.DMA for _ in range(2)],
            pltpu.VMEM((LANES,), jnp.float32),
        ],
    )
    def k(y_hbm, j_hbm, sub_hbm, at_hbm, ut_hbm, rt_hbm, out_hbm,
          y_v, j_v, idx_v, a_v, u_v, r_v, lsem, gsem, acc_v):
        wid = lax.axis_index("s") * NUM_CORES + lax.axis_index("c")
        base = wid * PER_W

        def start_linear(i, b):
            off = base + i * CHUNK
            pltpu.async_copy(y_hbm.at[pl.ds(off, CHUNK)], y_v[b], lsem[b])
            pltpu.async_copy(j_hbm.at[pl.ds(off, CHUNK)], j_v[b], lsem[b])
            pltpu.async_copy(sub_hbm.at[pl.ds(off, CHUNK)], idx_v[b], lsem[b])

        def drain_linear(i, b):
            off = base + i * CHUNK
            pltpu.make_async_copy(y_hbm.at[pl.ds(off, CHUNK)], y_v[b], lsem[b]).wait()
            pltpu.make_async_copy(j_hbm.at[pl.ds(off, CHUNK)], j_v[b], lsem[b]).wait()
            pltpu.make_async_copy(sub_hbm.at[pl.ds(off, CHUNK)], idx_v[b], lsem[b]).wait()

        def start_gather(b):
            pltpu.async_copy(at_hbm.at[idx_v[b]], a_v[b], gsem[b])
            pltpu.async_copy(ut_hbm.at[idx_v[b]], u_v[b], gsem[b])
            pltpu.async_copy(rt_hbm.at[idx_v[b]], r_v[b], gsem[b])

        def wait_gather(b):
            pltpu.make_async_copy(at_hbm.at[idx_v[b]], a_v[b], gsem[b]).wait()
            pltpu.make_async_copy(ut_hbm.at[idx_v[b]], u_v[b], gsem[b]).wait()
            pltpu.make_async_copy(rt_hbm.at[idx_v[b]], r_v[b], gsem[b]).wait()

        def compute(b, acc):
            def vec_body(v, acc):
                sl = pl.ds(v * LANES, LANES)
                mu = a_v[b][sl] - u_v[b][sl] * jnp.exp(-r_v[b][sl] * j_v[b][sl])
                r = y_v[b][sl] - mu
                return acc + r * r

            return lax.fori_loop(0, CHUNK // LANES, vec_body, acc)

        start_linear(0, 0)
        drain_linear(0, 0)
        start_gather(0)
        start_linear(1, 1)

        def one_iter(i, b, acc):
            @pl.when(i + 1 < NCHUNK)
            def _():
                drain_linear(i + 1, 1 - b)
                start_gather(1 - b)

            wait_gather(b)
            acc = compute(b, acc)

            @pl.when(i + 2 < NCHUNK)
            def _():
                start_linear(i + 2, b)

            return acc

        def pair_body(i2, acc):
            i = i2 * 2
            acc = one_iter(i, 0, acc)
            return one_iter(i + 1, 1, acc)

        acc = lax.fori_loop(0, NCHUNK // 2, pair_body,
                            jnp.zeros((LANES,), jnp.float32))
        acc_v[...] = acc
        pltpu.sync_copy(acc_v, out_hbm.at[wid])

    return k(y, j, sub, a_tab, u_tab, r_tab)


def kernel(y, j, sub, A, U, Lambda):
    pad = SUBJECTS_PAD - A.shape[0]
    a_tab, u_tab, r_tab = _transform_tables(
        jnp.pad(A, (0, pad)), jnp.pad(U, (0, pad)), jnp.pad(Lambda, (0, pad)))
    partials = _loss_partials(y, j, sub, a_tab, u_tab, r_tab)
    return jnp.sqrt(jnp.sum(partials) / N_OBS)


# speedup vs baseline: 632.3586x; 3.1064x over previous
import functools

import jax
import jax.numpy as jnp
from jax import lax
from jax.experimental import pallas as pl
from jax.experimental.pallas import tpu as pltpu
from jax.experimental.pallas import tpu_sc as plsc

N_OBS = 16384 * 200
NUM_CORES = 2
NUM_SUBCORES = 16
NW = NUM_CORES * NUM_SUBCORES
PER_W = N_OBS // NW
CHUNK = 2048
NCHUNK = PER_W // CHUNK
LANES = 16

SUBJECTS_PAD = 100352
ROWS_W = SUBJECTS_PAD // NW

_PARAMS = pltpu.CompilerParams(needs_layout_passes=False)
_HI = jnp.int32(-65536)


def _transform_tables(A, U, Lambda):
    mesh = plsc.VectorSubcoreMesh(core_axis_name="c", subcore_axis_name="s")

    @functools.partial(
        pl.kernel,
        out_type=(pltpu.HBM((SUBJECTS_PAD,), jnp.int32),
                  pltpu.HBM((SUBJECTS_PAD,), jnp.int32)),
        mesh=mesh,
        compiler_params=_PARAMS,
        scratch_types=[
            pltpu.VMEM((ROWS_W,), jnp.float32),
            pltpu.VMEM((ROWS_W,), jnp.float32),
            pltpu.VMEM((ROWS_W,), jnp.float32),
            pltpu.VMEM((ROWS_W,), jnp.int32),
            pltpu.VMEM((ROWS_W,), jnp.int32),
        ],
    )
    def k(a_hbm, u_hbm, l_hbm, w_hbm, r_hbm, a_v, u_v, l_v, w_v, r_v):
        wid = lax.axis_index("s") * NUM_CORES + lax.axis_index("c")
        base = wid * ROWS_W
        pltpu.sync_copy(a_hbm.at[pl.ds(base, ROWS_W)], a_v)
        pltpu.sync_copy(u_hbm.at[pl.ds(base, ROWS_W)], u_v)
        pltpu.sync_copy(l_hbm.at[pl.ds(base, ROWS_W)], l_v)

        def body(v, carry):
            sl = pl.ds(v * LANES, LANES)
            ai = lax.bitcast_convert_type(jnp.maximum(a_v[sl], 0.0), jnp.int32)
            ui = lax.bitcast_convert_type(jnp.maximum(u_v[sl], 0.0), jnp.int32)
            hi = (ai + 0x8000) & _HI
            lo = lax.shift_right_logical(ui + 0x8000, 16)
            w_v[sl] = hi | lo
            r_v[sl] = lax.bitcast_convert_type(
                0.2 / (1.0 + jnp.exp(-l_v[sl])), jnp.int32)
            return carry

        lax.fori_loop(0, ROWS_W // LANES, body, 0)
        pltpu.sync_copy(w_v, w_hbm.at[pl.ds(base, ROWS_W)])
        pltpu.sync_copy(r_v, r_hbm.at[pl.ds(base, ROWS_W)])

    return k(A, U, Lambda)


def _loss_partials(y, j, sub, w_tab, r_tab):
    mesh = plsc.VectorSubcoreMesh(core_axis_name="c", subcore_axis_name="s")

    @functools.partial(
        pl.kernel,
        out_type=(jax.ShapeDtypeStruct((NW, LANES), jnp.float32),
                  pltpu.HBM((N_OBS,), jnp.int32)),
        mesh=mesh,
        compiler_params=_PARAMS,
        scratch_types=[
            pltpu.VMEM((SUBJECTS_PAD,), jnp.int32),
            [pltpu.VMEM((CHUNK,), jnp.int32) for _ in range(2)],
            [pltpu.VMEM((CHUNK,), jnp.int32) for _ in range(2)],
            [pltpu.VMEM((CHUNK,), jnp.float32) for _ in range(2)],
            [pltpu.VMEM((CHUNK,), jnp.float32) for _ in range(2)],
            [pltpu.SemaphoreType.DMA for _ in range(2)],
            [pltpu.SemaphoreType.DMA for _ in range(2)],
            pltpu.VMEM((LANES,), jnp.float32),
        ],
    )
    def k(y_hbm, j_hbm, sub_hbm, wt_hbm, rt_hbm, out_hbm, ws_hbm,
          tab_v, idx_v, w_v, y_v, j_v, isem, osem, acc_v):
        wid = lax.axis_index("s") * NUM_CORES + lax.axis_index("c")
        base = wid * PER_W

        def chunk_at(hbm, i):
            return hbm.at[pl.ds(base + i * CHUNK, CHUNK)]

        pltpu.sync_copy(wt_hbm, tab_v)
        pltpu.async_copy(chunk_at(sub_hbm, 0), idx_v[0], isem[0])
        pltpu.async_copy(chunk_at(sub_hbm, 1), idx_v[1], isem[1])

        def p1_iter(i, b):
            pltpu.make_async_copy(chunk_at(sub_hbm, i), idx_v[b], isem[b]).wait()

            @pl.when(i >= 2)
            def _():
                pltpu.make_async_copy(w_v[b], chunk_at(ws_hbm, i - 2), osem[b]).wait()

            def gbody(v, carry):
                sl = pl.ds(v * LANES, LANES)
                w_v[b][sl] = plsc.load_gather(tab_v, [idx_v[b][sl]])
                return carry

            lax.fori_loop(0, CHUNK // LANES, gbody, 0)
            pltpu.async_copy(w_v[b], chunk_at(ws_hbm, i), osem[b])

            @pl.when(i + 2 < NCHUNK)
            def _():
                pltpu.async_copy(chunk_at(sub_hbm, i + 2), idx_v[b], isem[b])

        def p1_pair(i2, carry):
            p1_iter(i2 * 2, 0)
            p1_iter(i2 * 2 + 1, 1)
            return carry

        lax.fori_loop(0, NCHUNK // 2, p1_pair, 0)
        pltpu.make_async_copy(w_v[0], chunk_at(ws_hbm, NCHUNK - 2), osem[0]).wait()
        pltpu.make_async_copy(w_v[1], chunk_at(ws_hbm, NCHUNK - 1), osem[1]).wait()

        pltpu.sync_copy(rt_hbm, tab_v)

        def p2_start(i, b):
            pltpu.async_copy(chunk_at(y_hbm, i), y_v[b], isem[b])
            pltpu.async_copy(chunk_at(j_hbm, i), j_v[b], isem[b])
            pltpu.async_copy(chunk_at(sub_hbm, i), idx_v[b], isem[b])
            pltpu.async_copy(chunk_at(ws_hbm, i), w_v[b], isem[b])

        def p2_drain(i, b):
            pltpu.make_async_copy(chunk_at(y_hbm, i), y_v[b], isem[b]).wait()
            pltpu.make_async_copy(chunk_at(j_hbm, i), j_v[b], isem[b]).wait()
            pltpu.make_async_copy(chunk_at(sub_hbm, i), idx_v[b], isem[b]).wait()
            pltpu.make_async_copy(chunk_at(ws_hbm, i), w_v[b], isem[b]).wait()

        p2_start(0, 0)
        p2_start(1, 1)

        def p2_iter(i, b, acc):
            p2_drain(i, b)

            def vec_body(v, acc):
                sl = pl.ds(v * LANES, LANES)
                w = w_v[b][sl]
                a = lax.bitcast_convert_type(w & _HI, jnp.float32)
                u = lax.bitcast_convert_type(lax.shift_left(w, 16), jnp.float32)
                rate = lax.bitcast_convert_type(
                    plsc.load_gather(tab_v, [idx_v[b][sl]]), jnp.float32)
                mu = a - u * jnp.exp(-rate * j_v[b][sl])
                r = y_v[b][sl] - mu
                return acc + r * r

            acc = lax.fori_loop(0, CHUNK // LANES, vec_body, acc)

            @pl.when(i + 2 < NCHUNK)
            def _():
                p2_start(i + 2, b)

            return acc

        def p2_pair(i2, acc):
            acc = p2_iter(i2 * 2, 0, acc)
            return p2_iter(i2 * 2 + 1, 1, acc)

        acc = lax.fori_loop(0, NCHUNK // 2, p2_pair,
                            jnp.zeros((LANES,), jnp.float32))
        acc_v[...] = acc
        pltpu.sync_copy(acc_v, out_hbm.at[wid])

    return k(y, j, sub, w_tab, r_tab)


def kernel(y, j, sub, A, U, Lambda):
    pad = SUBJECTS_PAD - A.shape[0]
    w_tab, r_tab = _transform_tables(
        jnp.pad(A, (0, pad)), jnp.pad(U, (0, pad)), jnp.pad(Lambda, (0, pad)))
    partials, _ = _loss_partials(y, j, sub, w_tab, r_tab)
    return jnp.sqrt(jnp.sum(partials) / N_OBS)


# speedup vs baseline: 811.3598x; 1.2831x over previous
import functools

import jax
import jax.numpy as jnp
from jax import lax
from jax.experimental import pallas as pl
from jax.experimental.pallas import tpu as pltpu
from jax.experimental.pallas import tpu_sc as plsc

N_OBS = 16384 * 200
NUM_CORES = 2
NUM_SUBCORES = 16
NW = NUM_CORES * NUM_SUBCORES
PER_W = N_OBS // NW
CHUNK = 800
NCHUNK = PER_W // CHUNK
LANES = 16

SUBJECTS_PAD = 100352
ROWS_W = SUBJECTS_PAD // NW

_PARAMS = pltpu.CompilerParams(needs_layout_passes=False)
_HI = jnp.int32(-65536)
_RINV = 255.0 / 0.2
_RSC = 0.2 / 255.0


def _transform_tables(A, U, Lambda):
    mesh = plsc.VectorSubcoreMesh(core_axis_name="c", subcore_axis_name="s")

    @functools.partial(
        pl.kernel,
        out_type=(pltpu.HBM((SUBJECTS_PAD,), jnp.int32),
                  pltpu.HBM((SUBJECTS_PAD // 4,), jnp.int32)),
        mesh=mesh,
        compiler_params=_PARAMS,
        scratch_types=[
            pltpu.VMEM((ROWS_W,), jnp.float32),
            pltpu.VMEM((ROWS_W,), jnp.float32),
            pltpu.VMEM((ROWS_W,), jnp.float32),
            pltpu.VMEM((ROWS_W,), jnp.int32),
            pltpu.VMEM((ROWS_W,), jnp.int32),
            pltpu.VMEM((ROWS_W // 4,), jnp.int32),
        ],
    )
    def k(a_hbm, u_hbm, l_hbm, w_hbm, rq_hbm, a_v, u_v, l_v, w_v, q_v, rq_v):
        wid = lax.axis_index("s") * NUM_CORES + lax.axis_index("c")
        base = wid * ROWS_W
        pltpu.sync_copy(a_hbm.at[pl.ds(base, ROWS_W)], a_v)
        pltpu.sync_copy(u_hbm.at[pl.ds(base, ROWS_W)], u_v)
        pltpu.sync_copy(l_hbm.at[pl.ds(base, ROWS_W)], l_v)

        def body(v, carry):
            sl = pl.ds(v * LANES, LANES)
            ai = lax.bitcast_convert_type(jnp.maximum(a_v[sl], 0.0), jnp.int32)
            ui = lax.bitcast_convert_type(jnp.maximum(u_v[sl], 0.0), jnp.int32)
            hi = (ai + 0x8000) & _HI
            lo = lax.shift_right_logical(ui + 0x8000, 16)
            w_v[sl] = hi | lo
            rate = 0.2 / (1.0 + jnp.exp(-l_v[sl]))
            q_v[sl] = lax.convert_element_type(rate * _RINV + 0.5, jnp.int32)
            return carry

        lax.fori_loop(0, ROWS_W // LANES, body, 0)
        iota = lax.iota(jnp.int32, LANES)

        def pack_body(v, carry):
            byte0 = (v * LANES + iota) * 4
            g0 = plsc.load_gather(q_v, [byte0])
            g1 = plsc.load_gather(q_v, [byte0 + 1])
            g2 = plsc.load_gather(q_v, [byte0 + 2])
            g3 = plsc.load_gather(q_v, [byte0 + 3])
            rq_v[pl.ds(v * LANES, LANES)] = (
                g0 | lax.shift_left(g1, 8) | lax.shift_left(g2, 16)
                | lax.shift_left(g3, 24))
            return carry

        lax.fori_loop(0, ROWS_W // 4 // LANES, pack_body, 0)
        pltpu.sync_copy(w_v, w_hbm.at[pl.ds(base, ROWS_W)])
        pltpu.sync_copy(rq_v, rq_hbm.at[pl.ds(wid * (ROWS_W // 4), ROWS_W // 4)])

    return k(A, U, Lambda)


def _loss_partials(y, j, sub, w_tab, rq_tab):
    mesh = plsc.VectorSubcoreMesh(core_axis_name="c", subcore_axis_name="s")

    @functools.partial(
        pl.kernel,
        out_type=jax.ShapeDtypeStruct((NW, LANES), jnp.float32),
        mesh=mesh,
        compiler_params=_PARAMS,
        scratch_types=[
            pltpu.VMEM((SUBJECTS_PAD,), jnp.int32),
            pltpu.VMEM((SUBJECTS_PAD // 4,), jnp.int32),
            [pltpu.VMEM((CHUNK,), jnp.int32) for _ in range(2)],
            [pltpu.VMEM((CHUNK,), jnp.float32) for _ in range(2)],
            [pltpu.VMEM((CHUNK,), jnp.float32) for _ in range(2)],
            [pltpu.SemaphoreType.DMA for _ in range(2)],
            pltpu.VMEM((LANES,), jnp.float32),
        ],
    )
    def k(y_hbm, j_hbm, sub_hbm, wt_hbm, rqt_hbm, out_hbm,
          tab_v, tabr_v, idx_v, y_v, j_v, isem, acc_v):
        wid = lax.axis_index("s") * NUM_CORES + lax.axis_index("c")
        base = wid * PER_W

        def chunk_at(hbm, i):
            return hbm.at[pl.ds(base + i * CHUNK, CHUNK)]

        def start_in(i, b):
            pltpu.async_copy(chunk_at(y_hbm, i), y_v[b], isem[b])
            pltpu.async_copy(chunk_at(j_hbm, i), j_v[b], isem[b])
            pltpu.async_copy(chunk_at(sub_hbm, i), idx_v[b], isem[b])

        def drain_in(i, b):
            pltpu.make_async_copy(chunk_at(y_hbm, i), y_v[b], isem[b]).wait()
            pltpu.make_async_copy(chunk_at(j_hbm, i), j_v[b], isem[b]).wait()
            pltpu.make_async_copy(chunk_at(sub_hbm, i), idx_v[b], isem[b]).wait()

        start_in(0, 0)
        start_in(1, 1)
        pltpu.sync_copy(wt_hbm, tab_v)
        pltpu.sync_copy(rqt_hbm, tabr_v)

        def one_iter(i, b, acc):
            drain_in(i, b)

            def vec_body(v, acc):
                sl = pl.ds(v * LANES, LANES)
                s = idx_v[b][sl]
                w = plsc.load_gather(tab_v, [s])
                qw = plsc.load_gather(tabr_v, [lax.shift_right_logical(s, 2)])
                sh = lax.shift_left(s & 3, 3)
                q = lax.shift_right_logical(qw, sh) & 0xFF
                rate = lax.convert_element_type(q, jnp.float32) * _RSC
                a = lax.bitcast_convert_type(w & _HI, jnp.float32)
                u = lax.bitcast_convert_type(lax.shift_left(w, 16), jnp.float32)
                mu = a - u * jnp.exp(-rate * j_v[b][sl])
                r = y_v[b][sl] - mu
                return acc + r * r

            acc = lax.fori_loop(0, CHUNK // LANES, vec_body, acc)

            @pl.when(i + 2 < NCHUNK)
            def _():
                start_in(i + 2, b)

            return acc

        def pair_body(i2, acc):
            acc = one_iter(i2 * 2, 0, acc)
            return one_iter(i2 * 2 + 1, 1, acc)

        acc = lax.fori_loop(0, NCHUNK // 2, pair_body,
                            jnp.zeros((LANES,), jnp.float32))
        acc_v[...] = acc
        pltpu.sync_copy(acc_v, out_hbm.at[wid])

    return k(y, j, sub, w_tab, rq_tab)


def kernel(y, j, sub, A, U, Lambda):
    pad = SUBJECTS_PAD - A.shape[0]
    w_tab, rq_tab = _transform_tables(
        jnp.pad(A, (0, pad)), jnp.pad(U, (0, pad)), jnp.pad(Lambda, (0, pad)))
    partials = _loss_partials(y, j, sub, w_tab, rq_tab)
    return jnp.sqrt(jnp.sum(partials) / N_OBS)


# speedup vs baseline: 840.2038x; 1.0356x over previous
import functools

import jax
import jax.numpy as jnp
from jax import lax
from jax.experimental import pallas as pl
from jax.experimental.pallas import tpu as pltpu
from jax.experimental.pallas import tpu_sc as plsc

N_OBS = 16384 * 200
NUM_CORES = 2
NUM_SUBCORES = 16
NW = NUM_CORES * NUM_SUBCORES
PER_W = N_OBS // NW
CHUNK = 800
NCHUNK = PER_W // CHUNK
LANES = 16

SUBJECTS_PAD = 100352
ROWS_W = SUBJECTS_PAD // NW

_PARAMS = pltpu.CompilerParams(needs_layout_passes=False)
_HI = jnp.int32(-65536)
_RINV = 255.0 / 0.2
_RSC = 0.2 / 255.0
_NRSC = -0.2 / 255.0


def _transform_tables(A, U, Lambda):
    mesh = plsc.VectorSubcoreMesh(core_axis_name="c", subcore_axis_name="s")

    @functools.partial(
        pl.kernel,
        out_type=(pltpu.HBM((SUBJECTS_PAD,), jnp.int32),
                  pltpu.HBM((SUBJECTS_PAD // 4,), jnp.int32)),
        mesh=mesh,
        compiler_params=_PARAMS,
        scratch_types=[
            pltpu.VMEM((ROWS_W,), jnp.float32),
            pltpu.VMEM((ROWS_W,), jnp.float32),
            pltpu.VMEM((ROWS_W,), jnp.float32),
            pltpu.VMEM((ROWS_W,), jnp.int32),
            pltpu.VMEM((ROWS_W,), jnp.int32),
            pltpu.VMEM((ROWS_W // 4,), jnp.int32),
        ],
    )
    def k(a_hbm, u_hbm, l_hbm, w_hbm, rq_hbm, a_v, u_v, l_v, w_v, q_v, rq_v):
        wid = lax.axis_index("s") * NUM_CORES + lax.axis_index("c")
        base = wid * ROWS_W
        pltpu.sync_copy(a_hbm.at[pl.ds(base, ROWS_W)], a_v)
        pltpu.sync_copy(u_hbm.at[pl.ds(base, ROWS_W)], u_v)
        pltpu.sync_copy(l_hbm.at[pl.ds(base, ROWS_W)], l_v)

        def body(v, carry):
            sl = pl.ds(v * LANES, LANES)
            ai = lax.bitcast_convert_type(jnp.maximum(a_v[sl], 0.0), jnp.int32)
            ui = lax.bitcast_convert_type(jnp.maximum(u_v[sl], 0.0), jnp.int32)
            hi = (ai + 0x8000) & _HI
            lo = lax.shift_right_logical(ui + 0x8000, 16)
            w_v[sl] = hi | lo
            rate = 0.2 / (1.0 + jnp.exp(-l_v[sl]))
            q_v[sl] = lax.convert_element_type(rate * _RINV + 0.5, jnp.int32)
            return carry

        lax.fori_loop(0, ROWS_W // LANES, body, 0)
        iota = lax.iota(jnp.int32, LANES)

        def pack_body(v, carry):
            byte0 = (v * LANES + iota) * 4
            g0 = plsc.load_gather(q_v, [byte0])
            g1 = plsc.load_gather(q_v, [byte0 + 1])
            g2 = plsc.load_gather(q_v, [byte0 + 2])
            g3 = plsc.load_gather(q_v, [byte0 + 3])
            rq_v[pl.ds(v * LANES, LANES)] = (
                g0 | lax.shift_left(g1, 8) | lax.shift_left(g2, 16)
                | lax.shift_left(g3, 24))
            return carry

        lax.fori_loop(0, ROWS_W // 4 // LANES, pack_body, 0)
        pltpu.sync_copy(w_v, w_hbm.at[pl.ds(base, ROWS_W)])
        pltpu.sync_copy(rq_v, rq_hbm.at[pl.ds(wid * (ROWS_W // 4), ROWS_W // 4)])

    return k(A, U, Lambda)


def _loss_partials(y, j, sub, w_tab, rq_tab):
    mesh = plsc.VectorSubcoreMesh(core_axis_name="c", subcore_axis_name="s")

    @functools.partial(
        pl.kernel,
        out_type=jax.ShapeDtypeStruct((NW, LANES), jnp.float32),
        mesh=mesh,
        compiler_params=_PARAMS,
        scratch_types=[
            pltpu.VMEM((SUBJECTS_PAD,), jnp.int32),
            pltpu.VMEM((SUBJECTS_PAD // 4,), jnp.int32),
            [pltpu.VMEM((CHUNK,), jnp.int32) for _ in range(2)],
            [pltpu.VMEM((CHUNK,), jnp.float32) for _ in range(2)],
            [pltpu.VMEM((CHUNK,), jnp.float32) for _ in range(2)],
            [pltpu.SemaphoreType.DMA for _ in range(2)],
            pltpu.VMEM((LANES,), jnp.float32),
        ],
    )
    def k(y_hbm, j_hbm, sub_hbm, wt_hbm, rqt_hbm, out_hbm,
          tab_v, tabr_v, idx_v, y_v, j_v, isem, acc_v):
        wid = lax.axis_index("s") * NUM_CORES + lax.axis_index("c")
        base = wid * PER_W

        def chunk_at(hbm, i):
            return hbm.at[pl.ds(base + i * CHUNK, CHUNK)]

        def start_in(i, b):
            pltpu.async_copy(chunk_at(y_hbm, i), y_v[b], isem[b])
            pltpu.async_copy(chunk_at(j_hbm, i), j_v[b], isem[b])
            pltpu.async_copy(chunk_at(sub_hbm, i), idx_v[b], isem[b])

        def drain_in(i, b):
            pltpu.make_async_copy(chunk_at(y_hbm, i), y_v[b], isem[b]).wait()
            pltpu.make_async_copy(chunk_at(j_hbm, i), j_v[b], isem[b]).wait()
            pltpu.make_async_copy(chunk_at(sub_hbm, i), idx_v[b], isem[b]).wait()

        start_in(0, 0)
        start_in(1, 1)
        pltpu.sync_copy(wt_hbm, tab_v)
        pltpu.sync_copy(rqt_hbm, tabr_v)

        def one_iter(i, b, acc):
            drain_in(i, b)

            @plsc.parallel_loop(0, CHUNK // LANES, step=1, unroll=5, carry=acc)
            def vec_body(v, acc):
                sl = pl.ds(v * LANES, LANES)
                s = idx_v[b][sl]
                w = plsc.load_gather(tab_v, [s])
                qw = plsc.load_gather(tabr_v, [lax.shift_right_logical(s, 2)])
                sh = lax.shift_left(s & 3, 3)
                q = lax.shift_right_logical(qw, sh) & 0xFF
                nrate = lax.convert_element_type(q, jnp.float32) * _NRSC
                a = lax.bitcast_convert_type(w & _HI, jnp.float32)
                u = lax.bitcast_convert_type(lax.shift_left(w, 16), jnp.float32)
                mu = a - u * jnp.exp(nrate * j_v[b][sl])
                r = y_v[b][sl] - mu
                return acc + r * r

            acc = vec_body

            @pl.when(i + 2 < NCHUNK)
            def _():
                start_in(i + 2, b)

            return acc

        def pair_body(i2, acc):
            acc = one_iter(i2 * 2, 0, acc)
            return one_iter(i2 * 2 + 1, 1, acc)

        acc = lax.fori_loop(0, NCHUNK // 2, pair_body,
                            jnp.zeros((LANES,), jnp.float32))
        acc_v[...] = acc
        pltpu.sync_copy(acc_v, out_hbm.at[wid])

    return k(y, j, sub, w_tab, rq_tab)


def kernel(y, j, sub, A, U, Lambda):
    pad = SUBJECTS_PAD - A.shape[0]
    w_tab, rq_tab = _transform_tables(
        jnp.pad(A, (0, pad)), jnp.pad(U, (0, pad)), jnp.pad(Lambda, (0, pad)))
    partials = _loss_partials(y, j, sub, w_tab, rq_tab)
    return jnp.sqrt(jnp.sum(partials) / N_OBS)
